# Initial kernel scaffold; baseline (speedup 1.0000x reference)
import jax
import jax.numpy as jnp
from jax.experimental import pallas as pl


def kernel(edge_index, graph_ids, W1, b1, W2, b2, Wc1, bc1, Wc2, bc2, Wc3, bc3, Wc4, bc4, Wc5, bc5):
    raise NotImplementedError("write your pallas kernel here")


# speedup vs baseline: 10.0063x; 10.0063x over previous
import functools

import jax
import jax.numpy as jnp
from jax import lax
from jax.experimental import pallas as pl
from jax.experimental.pallas import tpu as pltpu
from jax.experimental.pallas import tpu_sc as plsc

F32 = jnp.float32

N = 10000
G = 128
HID = 512
NPAD = 10240
NSUB = 16
NODES_PER_TILE = NPAD // NSUB
EPAD = 655360
ROWS = EPAD // 128
ROWS_PER_TILE = ROWS // NSUB
CHUNK_ROWS = 8

@functools.cache
def _mesh():
    return plsc.VectorSubcoreMesh(core_axis_name="c", subcore_axis_name="s")


def _deg_body(srcp, dstp, out, idx_v, ones_v, zero_v, deg_sp):
    core = lax.axis_index("c")
    sub = lax.axis_index("s")
    for i in range(8):
        ones_v[0, pl.ds(i * 16, 16)] = jnp.ones((16,), F32)
    for i in range(NODES_PER_TILE // 16):
        zero_v[pl.ds(i * 16, 16)] = jnp.zeros((16,), F32)
    pltpu.sync_copy(zero_v, deg_sp.at[pl.ds(sub * NODES_PER_TILE, NODES_PER_TILE)])
    plsc.subcore_barrier()

    def run(idx_hbm):
        base = sub * ROWS_PER_TILE

        def chunk(ci, carry):
            row = base + ci * CHUNK_ROWS
            pltpu.sync_copy(idx_hbm.at[pl.ds(row, CHUNK_ROWS)], idx_v)
            for r in range(CHUNK_ROWS):
                pltpu.sync_copy(ones_v.at[0], deg_sp.at[idx_v.at[r]], add=True)
            return carry

        lax.fori_loop(0, ROWS_PER_TILE // CHUNK_ROWS, chunk, 0)

    @pl.when(core == 0)
    def _():
        run(dstp)

    @pl.when(core == 1)
    def _():
        run(srcp)

    plsc.subcore_barrier()
    off = core * NPAD + sub * NODES_PER_TILE
    pltpu.sync_copy(deg_sp.at[pl.ds(sub * NODES_PER_TILE, NODES_PER_TILE)],
                    out.at[pl.ds(off, NODES_PER_TILE)])


@functools.cache
def _deg_kernel():
    return pl.kernel(
        _deg_body,
        out_type=jax.ShapeDtypeStruct((2 * NPAD,), F32),
        mesh=_mesh(),
        scratch_types=[
            pltpu.VMEM((CHUNK_ROWS, 128), jnp.int32),
            pltpu.VMEM((1, 128), F32),
            pltpu.VMEM((NODES_PER_TILE,), F32),
            pltpu.VMEM_SHARED((NPAD,), F32),
        ],
    )


def _tsum_body(v_hbm, srcp, dstp, out, v_v, sidx_v, didx_v, contrib_v, zero_v, t_sp):
    core = lax.axis_index("c")
    sub = lax.axis_index("s")
    for i in range(NODES_PER_TILE // 16):
        zero_v[pl.ds(i * 16, 16)] = jnp.zeros((16,), F32)
    pltpu.sync_copy(zero_v, t_sp.at[pl.ds(sub * NODES_PER_TILE, NODES_PER_TILE)])
    pltpu.sync_copy(v_hbm, v_v)
    plsc.subcore_barrier()

    rows_here = ROWS // (2 * NSUB)
    base = (core * NSUB + sub) * rows_here

    def chunk(ci, carry):
        row = base + ci * CHUNK_ROWS
        pltpu.sync_copy(srcp.at[pl.ds(row, CHUNK_ROWS)], sidx_v)
        pltpu.sync_copy(dstp.at[pl.ds(row, CHUNK_ROWS)], didx_v)
        for r in range(CHUNK_ROWS):
            for c in range(8):
                s16 = sidx_v[r, pl.ds(c * 16, 16)]
                contrib_v[r, pl.ds(c * 16, 16)] = plsc.load_gather(v_v, [s16])
        for r in range(CHUNK_ROWS):
            pltpu.sync_copy(contrib_v.at[r], t_sp.at[didx_v.at[r]], add=True)
        return carry

    lax.fori_loop(0, rows_here // CHUNK_ROWS, chunk, 0)

    plsc.subcore_barrier()
    off = core * NPAD + sub * NODES_PER_TILE
    pltpu.sync_copy(t_sp.at[pl.ds(sub * NODES_PER_TILE, NODES_PER_TILE)],
                    out.at[pl.ds(off, NODES_PER_TILE)])


@functools.cache
def _tsum_kernel():
    return pl.kernel(
        _tsum_body,
        out_type=jax.ShapeDtypeStruct((2 * NPAD,), F32),
        mesh=_mesh(),
        compiler_params=pltpu.CompilerParams(needs_layout_passes=False),
        scratch_types=[
            pltpu.VMEM((NPAD,), F32),
            pltpu.VMEM((CHUNK_ROWS, 128), jnp.int32),
            pltpu.VMEM((CHUNK_ROWS, 128), jnp.int32),
            pltpu.VMEM((CHUNK_ROWS, 128), F32),
            pltpu.VMEM((NODES_PER_TILE,), F32),
            pltpu.VMEM_SHARED((NPAD,), F32),
        ],
    )


def _agg_body(y_hbm, srcoff, dstp, out, sidx_v, didx_v, rows_v, zero_v, acc_sp, sem):
    core = lax.axis_index("c")
    sub = lax.axis_index("s")
    for i in range((64 * 128) // 16):
        zr = i // 8
        zc = (i % 8) * 16
        zero_v[zr, pl.ds(zc, 16)] = jnp.zeros((16,), F32)

    for j in range(2):
        p = core * 2 + j
        for k in range(NODES_PER_TILE // 64):
            pltpu.sync_copy(zero_v, acc_sp.at[pl.ds(sub * NODES_PER_TILE + k * 64, 64)])
        plsc.subcore_barrier()

        base = sub * ROWS_PER_TILE

        def chunk(ci, carry):
            row = base + ci * CHUNK_ROWS
            pltpu.sync_copy(srcoff.at[pl.ds(p * ROWS + row, CHUNK_ROWS)], sidx_v)
            pltpu.sync_copy(dstp.at[pl.ds(row, CHUNK_ROWS)], didx_v)
            for r in range(CHUNK_ROWS):
                pltpu.async_copy(y_hbm.at[sidx_v.at[r]], rows_v, sem).wait()
                pltpu.sync_copy(rows_v, acc_sp.at[didx_v.at[r]], add=True)
            return carry

        lax.fori_loop(0, ROWS_PER_TILE // CHUNK_ROWS, chunk, 0)

        plsc.subcore_barrier()
        pltpu.sync_copy(acc_sp.at[pl.ds(sub * NODES_PER_TILE, NODES_PER_TILE)],
                        out.at[pl.ds(p * NPAD + sub * NODES_PER_TILE, NODES_PER_TILE)])


@functools.cache
def _agg_kernel():
    return pl.kernel(
        _agg_body,
        out_type=jax.ShapeDtypeStruct((4 * NPAD, 128), F32),
        mesh=_mesh(),
        scratch_types=[
            pltpu.VMEM((CHUNK_ROWS, 128), jnp.int32),
            pltpu.VMEM((CHUNK_ROWS, 128), jnp.int32),
            pltpu.VMEM((128, 128), F32),
            pltpu.VMEM((64, 128), F32),
            pltpu.VMEM_SHARED((NPAD, 128), F32),
            pltpu.SemaphoreType.DMA,
        ],
    )


def _norms_body(din_ref, dout_ref, ns_ref, nd_ref, v_ref):
    din = din_ref[...]
    dout = dout_ref[...]
    ns = lax.rsqrt(jnp.maximum(dout, 1.0))
    nd = lax.rsqrt(jnp.maximum(din, 1.0))
    ns_ref[...] = ns
    nd_ref[...] = nd
    v_ref[...] = ns * din


def _tc_norms(deg_in, deg_out):
    return pl.pallas_call(
        _norms_body,
        out_shape=[jax.ShapeDtypeStruct((NPAD, 1), F32)] * 3,
    )(deg_in, deg_out)


def _h1y_body(t2_ref, nd_ref, ns_ref, w1_ref, b1_ref, w2_ref, y_ref):
    t = t2_ref[0] + t2_ref[1]
    u = t * nd_ref[...]
    h1 = jnp.maximum(u * w1_ref[...] + b1_ref[...], 0.0)
    h1s = h1 * ns_ref[...]
    y_ref[...] = jnp.dot(h1s, w2_ref[...], preferred_element_type=F32)


def _tc_h1y(t2, nd, ns, w1, b1, w2):
    nr = NPAD // 1024
    return pl.pallas_call(
        _h1y_body,
        grid=(nr, 4),
        in_specs=[
            pl.BlockSpec((2, 1024, 1), lambda r, p: (0, r, 0)),
            pl.BlockSpec((1024, 1), lambda r, p: (r, 0)),
            pl.BlockSpec((1024, 1), lambda r, p: (r, 0)),
            pl.BlockSpec((1, HID), lambda r, p: (0, 0)),
            pl.BlockSpec((1, HID), lambda r, p: (0, 0)),
            pl.BlockSpec((HID, 128), lambda r, p: (0, p)),
        ],
        out_specs=pl.BlockSpec((1024, 128), lambda r, p: (p * nr + r, 0)),
        out_shape=jax.ShapeDtypeStruct((4 * NPAD, 128), F32),
    )(t2, nd, ns, w1, b1, w2)


def _pool_body(agg_ref, nd_ref, b2_ref, gid_ref, pooled_ref, cnt_ref):
    r = pl.program_id(0)
    agg = jnp.concatenate([agg_ref[i] for i in range(4)], axis=1)
    h2 = jnp.maximum(agg * nd_ref[...] + b2_ref[...], 0.0)
    gid = gid_ref[...]
    pt = (lax.broadcasted_iota(jnp.int32, (G, 1024), 0)
          == gid).astype(F32)
    pp = jnp.dot(pt, h2, preferred_element_type=F32)
    pc = jnp.dot(pt, jnp.ones((1024, 1), F32), preferred_element_type=F32)

    @pl.when(r == 0)
    def _():
        pooled_ref[...] = pp
        cnt_ref[...] = pc

    @pl.when(r > 0)
    def _():
        pooled_ref[...] += pp
        cnt_ref[...] += pc


def _tc_pool(agg4, nd, b2, gid_row):
    nr = NPAD // 1024
    return pl.pallas_call(
        _pool_body,
        grid=(nr,),
        in_specs=[
            pl.BlockSpec((4, 1024, 128), lambda r: (0, r, 0)),
            pl.BlockSpec((1024, 1), lambda r: (r, 0)),
            pl.BlockSpec((1, HID), lambda r: (0, 0)),
            pl.BlockSpec((1, 1024), lambda r: (0, r)),
        ],
        out_specs=[
            pl.BlockSpec((G, HID), lambda r: (0, 0)),
            pl.BlockSpec((G, 1), lambda r: (0, 0)),
        ],
        out_shape=[
            jax.ShapeDtypeStruct((G, HID), F32),
            jax.ShapeDtypeStruct((G, 1), F32),
        ],
    )(agg4, nd, b2, gid_row)


def _mlp_body(pooled_ref, cnt_ref, w1, b1, w2, b2, w3, b3, w4, b4, w5, b5, out_ref):
    hg = pooled_ref[...] / jnp.maximum(cnt_ref[...], 1.0)
    x = jnp.maximum(jnp.dot(hg, w1[...], preferred_element_type=F32) + b1[...], 0.0)
    x = jnp.maximum(jnp.dot(x, w2[...], preferred_element_type=F32) + b2[...], 0.0)
    x = jnp.maximum(jnp.dot(x, w3[...], preferred_element_type=F32) + b3[...], 0.0)
    x = jnp.maximum(jnp.dot(x, w4[...], preferred_element_type=F32) + b4[...], 0.0)
    logits = jnp.dot(x, w5[...], preferred_element_type=F32) + b5[...]
    m = jnp.max(logits, axis=-1, keepdims=True)
    e = jnp.exp(logits - m)
    out_ref[...] = e / jnp.sum(e, axis=-1, keepdims=True)


def _tc_mlp(pooled, cnt, wc1, bc1, wc2, bc2, wc3, bc3, wc4, bc4, wc5, bc5):
    return pl.pallas_call(
        _mlp_body,
        out_shape=jax.ShapeDtypeStruct((G, 10), F32),
    )(pooled, cnt, wc1, bc1, wc2, bc2, wc3, bc3, wc4, bc4, wc5, bc5)


def kernel(edge_index, graph_ids, W1, b1, W2, b2,
           Wc1, bc1, Wc2, bc2, Wc3, bc3, Wc4, bc4, Wc5, bc5):
    E = edge_index.shape[1]
    src = edge_index[0]
    dst = edge_index[1]
    pad = N + (jnp.arange(EPAD - E, dtype=jnp.int32) % (NPAD - N))
    srcp = jnp.concatenate([src, pad]).reshape(ROWS, 128)
    dstp = jnp.concatenate([dst, pad]).reshape(ROWS, 128)
    srcoff = (srcp[None] + (jnp.arange(4, dtype=jnp.int32) * NPAD)[:, None, None]
              ).reshape(4 * ROWS, 128)

    deg = _deg_kernel()(srcp, dstp)
    deg_in = deg[:NPAD].reshape(NPAD, 1)
    deg_out = deg[NPAD:].reshape(NPAD, 1)

    ns, nd, v = _tc_norms(deg_in, deg_out)

    t2 = _tsum_kernel()(v.reshape(NPAD), srcp, dstp).reshape(2, NPAD, 1)

    y = _tc_h1y(t2, nd, ns, W1, b1.reshape(1, HID), W2)

    agg = _agg_kernel()(y, srcoff, dstp).reshape(4, NPAD, 128)

    gid_row = jnp.concatenate(
        [graph_ids, jnp.full((NPAD - N,), 1000, jnp.int32)]).reshape(1, NPAD)
    pooled, cnt = _tc_pool(agg, nd, b2.reshape(1, HID), gid_row)

    return _tc_mlp(pooled, cnt,
                   Wc1, bc1.reshape(1, -1), Wc2, bc2.reshape(1, -1),
                   Wc3, bc3.reshape(1, -1), Wc4, bc4.reshape(1, -1),
                   Wc5, bc5.reshape(1, -1))


# speedup vs baseline: 15.0336x; 1.5024x over previous
import functools

import jax
import jax.numpy as jnp
from jax import lax
from jax.experimental import pallas as pl
from jax.experimental.pallas import tpu as pltpu
from jax.experimental.pallas import tpu_sc as plsc

F32 = jnp.float32

N = 10000
G = 128
HID = 512
NPAD = 10240
NSUB = 16
NODES_PER_TILE = NPAD // NSUB
EPAD = 655360
ROWS = EPAD // 128
ROWS_PER_TILE = ROWS // NSUB
CHUNK_ROWS = 8

@functools.cache
def _mesh():
    return plsc.VectorSubcoreMesh(core_axis_name="c", subcore_axis_name="s")


def _deg_body(srcp, dstp, out, idx_v, ones_v, zero_v, deg_sp):
    core = lax.axis_index("c")
    sub = lax.axis_index("s")
    for i in range(8):
        ones_v[0, pl.ds(i * 16, 16)] = jnp.ones((16,), F32)
    for i in range(NODES_PER_TILE // 16):
        zero_v[pl.ds(i * 16, 16)] = jnp.zeros((16,), F32)
    pltpu.sync_copy(zero_v, deg_sp.at[pl.ds(sub * NODES_PER_TILE, NODES_PER_TILE)])
    plsc.subcore_barrier()

    def run(idx_hbm):
        base = sub * ROWS_PER_TILE

        def chunk(ci, carry):
            row = base + ci * CHUNK_ROWS
            pltpu.sync_copy(idx_hbm.at[pl.ds(row, CHUNK_ROWS)], idx_v)
            for r in range(CHUNK_ROWS):
                pltpu.sync_copy(ones_v.at[0], deg_sp.at[idx_v.at[r]], add=True)
            return carry

        lax.fori_loop(0, ROWS_PER_TILE // CHUNK_ROWS, chunk, 0)

    @pl.when(core == 0)
    def _():
        run(dstp)

    @pl.when(core == 1)
    def _():
        run(srcp)

    plsc.subcore_barrier()
    off = core * NPAD + sub * NODES_PER_TILE
    pltpu.sync_copy(deg_sp.at[pl.ds(sub * NODES_PER_TILE, NODES_PER_TILE)],
                    out.at[pl.ds(off, NODES_PER_TILE)])


@functools.cache
def _deg_kernel():
    return pl.kernel(
        _deg_body,
        out_type=jax.ShapeDtypeStruct((2 * NPAD,), F32),
        mesh=_mesh(),
        scratch_types=[
            pltpu.VMEM((CHUNK_ROWS, 128), jnp.int32),
            pltpu.VMEM((1, 128), F32),
            pltpu.VMEM((NODES_PER_TILE,), F32),
            pltpu.VMEM_SHARED((NPAD,), F32),
        ],
    )


def _tsum_body(v_hbm, srcp, dstp, out, v_v, sidx_v, didx_v, contrib_v, zero_v, t_sp):
    core = lax.axis_index("c")
    sub = lax.axis_index("s")
    for i in range(NODES_PER_TILE // 16):
        zero_v[pl.ds(i * 16, 16)] = jnp.zeros((16,), F32)
    pltpu.sync_copy(zero_v, t_sp.at[pl.ds(sub * NODES_PER_TILE, NODES_PER_TILE)])
    pltpu.sync_copy(v_hbm, v_v)
    plsc.subcore_barrier()

    rows_here = ROWS // (2 * NSUB)
    base = (core * NSUB + sub) * rows_here

    def chunk(ci, carry):
        row = base + ci * CHUNK_ROWS
        pltpu.sync_copy(srcp.at[pl.ds(row, CHUNK_ROWS)], sidx_v)
        pltpu.sync_copy(dstp.at[pl.ds(row, CHUNK_ROWS)], didx_v)
        for r in range(CHUNK_ROWS):
            for c in range(8):
                s16 = sidx_v[r, pl.ds(c * 16, 16)]
                contrib_v[r, pl.ds(c * 16, 16)] = plsc.load_gather(v_v, [s16])
        for r in range(CHUNK_ROWS):
            pltpu.sync_copy(contrib_v.at[r], t_sp.at[didx_v.at[r]], add=True)
        return carry

    lax.fori_loop(0, rows_here // CHUNK_ROWS, chunk, 0)

    plsc.subcore_barrier()
    off = core * NPAD + sub * NODES_PER_TILE
    pltpu.sync_copy(t_sp.at[pl.ds(sub * NODES_PER_TILE, NODES_PER_TILE)],
                    out.at[pl.ds(off, NODES_PER_TILE)])


@functools.cache
def _tsum_kernel():
    return pl.kernel(
        _tsum_body,
        out_type=jax.ShapeDtypeStruct((2 * NPAD,), F32),
        mesh=_mesh(),
        compiler_params=pltpu.CompilerParams(needs_layout_passes=False),
        scratch_types=[
            pltpu.VMEM((NPAD,), F32),
            pltpu.VMEM((CHUNK_ROWS, 128), jnp.int32),
            pltpu.VMEM((CHUNK_ROWS, 128), jnp.int32),
            pltpu.VMEM((CHUNK_ROWS, 128), F32),
            pltpu.VMEM((NODES_PER_TILE,), F32),
            pltpu.VMEM_SHARED((NPAD,), F32),
        ],
    )


ACHUNK = 16


def _agg_body(y_hbm, srcoff, dstp, out, sidx_v, didx_v, rows0, rows1,
              zero_v, acc_sp, sem0, sem1):
    core = lax.axis_index("c")
    sub = lax.axis_index("s")
    bufs = (rows0, rows1)
    sems = (sem0, sem1)
    for i in range((64 * 128) // 16):
        zr = i // 8
        zc = (i % 8) * 16
        zero_v[zr, pl.ds(zc, 16)] = jnp.zeros((16,), F32)

    for j in range(2):
        p = core * 2 + j
        for k in range(NODES_PER_TILE // 64):
            pltpu.sync_copy(zero_v, acc_sp.at[pl.ds(sub * NODES_PER_TILE + k * 64, 64)])
        plsc.subcore_barrier()

        base = sub * ROWS_PER_TILE

        def chunk(ci, carry):
            row = base + ci * ACHUNK
            pltpu.sync_copy(srcoff.at[pl.ds(p * ROWS + row, ACHUNK)], sidx_v)
            pltpu.sync_copy(dstp.at[pl.ds(row, ACHUNK)], didx_v)
            prev = pltpu.async_copy(y_hbm.at[sidx_v.at[0]], rows0, sem0)
            for r in range(ACHUNK):
                if r + 1 < ACHUNK:
                    nxt = pltpu.async_copy(y_hbm.at[sidx_v.at[r + 1]],
                                           bufs[(r + 1) % 2], sems[(r + 1) % 2])
                prev.wait()
                pltpu.sync_copy(bufs[r % 2], acc_sp.at[didx_v.at[r]], add=True)
                if r + 1 < ACHUNK:
                    prev = nxt
            return carry

        lax.fori_loop(0, ROWS_PER_TILE // ACHUNK, chunk, 0)

        plsc.subcore_barrier()
        pltpu.sync_copy(acc_sp.at[pl.ds(sub * NODES_PER_TILE, NODES_PER_TILE)],
                        out.at[pl.ds(p * NPAD + sub * NODES_PER_TILE, NODES_PER_TILE)])


@functools.cache
def _agg_kernel():
    return pl.kernel(
        _agg_body,
        out_type=jax.ShapeDtypeStruct((4 * NPAD, 128), F32),
        mesh=_mesh(),
        scratch_types=[
            pltpu.VMEM((ACHUNK, 128), jnp.int32),
            pltpu.VMEM((ACHUNK, 128), jnp.int32),
            pltpu.VMEM((128, 128), F32),
            pltpu.VMEM((128, 128), F32),
            pltpu.VMEM((64, 128), F32),
            pltpu.VMEM_SHARED((NPAD, 128), F32),
            pltpu.SemaphoreType.DMA,
            pltpu.SemaphoreType.DMA,
        ],
    )


def _norms_body(din_ref, dout_ref, ns_ref, nd_ref, v_ref):
    din = din_ref[...]
    dout = dout_ref[...]
    ns = lax.rsqrt(jnp.maximum(dout, 1.0))
    nd = lax.rsqrt(jnp.maximum(din, 1.0))
    ns_ref[...] = ns
    nd_ref[...] = nd
    v_ref[...] = ns * din


def _tc_norms(deg_in, deg_out):
    return pl.pallas_call(
        _norms_body,
        out_shape=[jax.ShapeDtypeStruct((NPAD, 1), F32)] * 3,
    )(deg_in, deg_out)


def _h1y_body(t2_ref, nd_ref, ns_ref, w1_ref, b1_ref, w2_ref, y_ref):
    t = t2_ref[0] + t2_ref[1]
    u = t * nd_ref[...]
    h1 = jnp.maximum(u * w1_ref[...] + b1_ref[...], 0.0)
    h1s = h1 * ns_ref[...]
    y_ref[...] = jnp.dot(h1s, w2_ref[...], preferred_element_type=F32)


def _tc_h1y(t2, nd, ns, w1, b1, w2):
    nr = NPAD // 1024
    return pl.pallas_call(
        _h1y_body,
        grid=(nr, 4),
        in_specs=[
            pl.BlockSpec((2, 1024, 1), lambda r, p: (0, r, 0)),
            pl.BlockSpec((1024, 1), lambda r, p: (r, 0)),
            pl.BlockSpec((1024, 1), lambda r, p: (r, 0)),
            pl.BlockSpec((1, HID), lambda r, p: (0, 0)),
            pl.BlockSpec((1, HID), lambda r, p: (0, 0)),
            pl.BlockSpec((HID, 128), lambda r, p: (0, p)),
        ],
        out_specs=pl.BlockSpec((1024, 128), lambda r, p: (p * nr + r, 0)),
        out_shape=jax.ShapeDtypeStruct((4 * NPAD, 128), F32),
    )(t2, nd, ns, w1, b1, w2)


def _pool_body(agg_ref, nd_ref, b2_ref, gid_ref, pooled_ref, cnt_ref):
    r = pl.program_id(0)
    agg = jnp.concatenate([agg_ref[i] for i in range(4)], axis=1)
    h2 = jnp.maximum(agg * nd_ref[...] + b2_ref[...], 0.0)
    gid = gid_ref[...]
    pt = (lax.broadcasted_iota(jnp.int32, (G, 1024), 0)
          == gid).astype(F32)
    pp = jnp.dot(pt, h2, preferred_element_type=F32)
    pc = jnp.dot(pt, jnp.ones((1024, 1), F32), preferred_element_type=F32)

    @pl.when(r == 0)
    def _():
        pooled_ref[...] = pp
        cnt_ref[...] = pc

    @pl.when(r > 0)
    def _():
        pooled_ref[...] += pp
        cnt_ref[...] += pc


def _tc_pool(agg4, nd, b2, gid_row):
    nr = NPAD // 1024
    return pl.pallas_call(
        _pool_body,
        grid=(nr,),
        in_specs=[
            pl.BlockSpec((4, 1024, 128), lambda r: (0, r, 0)),
            pl.BlockSpec((1024, 1), lambda r: (r, 0)),
            pl.BlockSpec((1, HID), lambda r: (0, 0)),
            pl.BlockSpec((1, 1024), lambda r: (0, r)),
        ],
        out_specs=[
            pl.BlockSpec((G, HID), lambda r: (0, 0)),
            pl.BlockSpec((G, 1), lambda r: (0, 0)),
        ],
        out_shape=[
            jax.ShapeDtypeStruct((G, HID), F32),
            jax.ShapeDtypeStruct((G, 1), F32),
        ],
    )(agg4, nd, b2, gid_row)


def _mlp_body(pooled_ref, cnt_ref, w1, b1, w2, b2, w3, b3, w4, b4, w5, b5, out_ref):
    hg = pooled_ref[...] / jnp.maximum(cnt_ref[...], 1.0)
    x = jnp.maximum(jnp.dot(hg, w1[...], preferred_element_type=F32) + b1[...], 0.0)
    x = jnp.maximum(jnp.dot(x, w2[...], preferred_element_type=F32) + b2[...], 0.0)
    x = jnp.maximum(jnp.dot(x, w3[...], preferred_element_type=F32) + b3[...], 0.0)
    x = jnp.maximum(jnp.dot(x, w4[...], preferred_element_type=F32) + b4[...], 0.0)
    logits = jnp.dot(x, w5[...], preferred_element_type=F32) + b5[...]
    m = jnp.max(logits, axis=-1, keepdims=True)
    e = jnp.exp(logits - m)
    out_ref[...] = e / jnp.sum(e, axis=-1, keepdims=True)


def _tc_mlp(pooled, cnt, wc1, bc1, wc2, bc2, wc3, bc3, wc4, bc4, wc5, bc5):
    return pl.pallas_call(
        _mlp_body,
        out_shape=jax.ShapeDtypeStruct((G, 10), F32),
    )(pooled, cnt, wc1, bc1, wc2, bc2, wc3, bc3, wc4, bc4, wc5, bc5)


def kernel(edge_index, graph_ids, W1, b1, W2, b2,
           Wc1, bc1, Wc2, bc2, Wc3, bc3, Wc4, bc4, Wc5, bc5):
    E = edge_index.shape[1]
    src = edge_index[0]
    dst = edge_index[1]
    pad = N + (jnp.arange(EPAD - E, dtype=jnp.int32) % (NPAD - N))
    srcp = jnp.concatenate([src, pad]).reshape(ROWS, 128)
    dstp = jnp.concatenate([dst, pad]).reshape(ROWS, 128)
    srcoff = (srcp[None] + (jnp.arange(4, dtype=jnp.int32) * NPAD)[:, None, None]
              ).reshape(4 * ROWS, 128)

    deg = _deg_kernel()(srcp, dstp)
    deg_in = deg[:NPAD].reshape(NPAD, 1)
    deg_out = deg[NPAD:].reshape(NPAD, 1)

    ns, nd, v = _tc_norms(deg_in, deg_out)

    t2 = _tsum_kernel()(v.reshape(NPAD), srcp, dstp).reshape(2, NPAD, 1)

    y = _tc_h1y(t2, nd, ns, W1, b1.reshape(1, HID), W2)

    agg = _agg_kernel()(y, srcoff, dstp).reshape(4, NPAD, 128)

    gid_row = jnp.concatenate(
        [graph_ids, jnp.full((NPAD - N,), 1000, jnp.int32)]).reshape(1, NPAD)
    pooled, cnt = _tc_pool(agg, nd, b2.reshape(1, HID), gid_row)

    return _tc_mlp(pooled, cnt,
                   Wc1, bc1.reshape(1, -1), Wc2, bc2.reshape(1, -1),
                   Wc3, bc3.reshape(1, -1), Wc4, bc4.reshape(1, -1),
                   Wc5, bc5.reshape(1, -1))


# speedup vs baseline: 15.9471x; 1.0608x over previous
import functools

import jax
import jax.numpy as jnp
from jax import lax
from jax.experimental import pallas as pl
from jax.experimental.pallas import tpu as pltpu
from jax.experimental.pallas import tpu_sc as plsc

F32 = jnp.float32

N = 10000
G = 128
HID = 512
NPAD = 10240
NSUB = 16
NODES_PER_TILE = NPAD // NSUB
EPAD = 655360
ROWS = EPAD // 128
ROWS_PER_TILE = ROWS // NSUB
CHUNK_ROWS = 8

@functools.cache
def _mesh():
    return plsc.VectorSubcoreMesh(core_axis_name="c", subcore_axis_name="s")


def _deg_body(srcp, dstp, out, idx_v, ones_v, zero_v, deg_sp):
    core = lax.axis_index("c")
    sub = lax.axis_index("s")
    for i in range(8):
        ones_v[0, pl.ds(i * 16, 16)] = jnp.ones((16,), F32)
    for i in range(NODES_PER_TILE // 16):
        zero_v[pl.ds(i * 16, 16)] = jnp.zeros((16,), F32)
    pltpu.sync_copy(zero_v, deg_sp.at[pl.ds(sub * NODES_PER_TILE, NODES_PER_TILE)])
    plsc.subcore_barrier()

    def run(idx_hbm):
        base = sub * ROWS_PER_TILE

        def chunk(ci, carry):
            row = base + ci * CHUNK_ROWS
            pltpu.sync_copy(idx_hbm.at[pl.ds(row, CHUNK_ROWS)], idx_v)
            for r in range(CHUNK_ROWS):
                pltpu.sync_copy(ones_v.at[0], deg_sp.at[idx_v.at[r]], add=True)
            return carry

        lax.fori_loop(0, ROWS_PER_TILE // CHUNK_ROWS, chunk, 0)

    @pl.when(core == 0)
    def _():
        run(dstp)

    @pl.when(core == 1)
    def _():
        run(srcp)

    plsc.subcore_barrier()
    off = core * NPAD + sub * NODES_PER_TILE
    pltpu.sync_copy(deg_sp.at[pl.ds(sub * NODES_PER_TILE, NODES_PER_TILE)],
                    out.at[pl.ds(off, NODES_PER_TILE)])


@functools.cache
def _deg_kernel():
    return pl.kernel(
        _deg_body,
        out_type=jax.ShapeDtypeStruct((2 * NPAD,), F32),
        mesh=_mesh(),
        scratch_types=[
            pltpu.VMEM((CHUNK_ROWS, 128), jnp.int32),
            pltpu.VMEM((1, 128), F32),
            pltpu.VMEM((NODES_PER_TILE,), F32),
            pltpu.VMEM_SHARED((NPAD,), F32),
        ],
    )


def _tsum_body(v_hbm, srcp, dstp, out, v_v, sidx_v, didx_v, contrib_v, zero_v, t_sp):
    core = lax.axis_index("c")
    sub = lax.axis_index("s")
    for i in range(NODES_PER_TILE // 16):
        zero_v[pl.ds(i * 16, 16)] = jnp.zeros((16,), F32)
    pltpu.sync_copy(zero_v, t_sp.at[pl.ds(sub * NODES_PER_TILE, NODES_PER_TILE)])
    pltpu.sync_copy(v_hbm, v_v)
    plsc.subcore_barrier()

    rows_here = ROWS // (2 * NSUB)
    base = (core * NSUB + sub) * rows_here

    def chunk(ci, carry):
        row = base + ci * CHUNK_ROWS
        pltpu.sync_copy(srcp.at[pl.ds(row, CHUNK_ROWS)], sidx_v)
        pltpu.sync_copy(dstp.at[pl.ds(row, CHUNK_ROWS)], didx_v)
        for r in range(CHUNK_ROWS):
            for c in range(8):
                s16 = sidx_v[r, pl.ds(c * 16, 16)]
                contrib_v[r, pl.ds(c * 16, 16)] = plsc.load_gather(v_v, [s16])
        for r in range(CHUNK_ROWS):
            pltpu.sync_copy(contrib_v.at[r], t_sp.at[didx_v.at[r]], add=True)
        return carry

    lax.fori_loop(0, rows_here // CHUNK_ROWS, chunk, 0)

    plsc.subcore_barrier()
    off = core * NPAD + sub * NODES_PER_TILE
    pltpu.sync_copy(t_sp.at[pl.ds(sub * NODES_PER_TILE, NODES_PER_TILE)],
                    out.at[pl.ds(off, NODES_PER_TILE)])


@functools.cache
def _tsum_kernel():
    return pl.kernel(
        _tsum_body,
        out_type=jax.ShapeDtypeStruct((2 * NPAD,), F32),
        mesh=_mesh(),
        compiler_params=pltpu.CompilerParams(needs_layout_passes=False),
        scratch_types=[
            pltpu.VMEM((NPAD,), F32),
            pltpu.VMEM((CHUNK_ROWS, 128), jnp.int32),
            pltpu.VMEM((CHUNK_ROWS, 128), jnp.int32),
            pltpu.VMEM((CHUNK_ROWS, 128), F32),
            pltpu.VMEM((NODES_PER_TILE,), F32),
            pltpu.VMEM_SHARED((NPAD,), F32),
        ],
    )


ACHUNK = 32
NCHUNK = ROWS_PER_TILE // ACHUNK
IBLK = 2 * ACHUNK


def _agg_body(y_hbm, edg, out, ib0, ib1, r0, r1, r2, r3, zero_v, acc_sp,
              g0, g1, g2, g3, is0, is1):
    core = lax.axis_index("c")
    sub = lax.axis_index("s")
    rows = (r0, r1, r2, r3)
    gs = (g0, g1, g2, g3)
    ibs = (ib0, ib1)
    iss = (is0, is1)
    for i in range((64 * 128) // 16):
        zr = i // 8
        zc = (i % 8) * 16
        zero_v[zr, pl.ds(zc, 16)] = jnp.zeros((16,), F32)

    for j in range(2):
        p = core * 2 + j
        for k in range(NODES_PER_TILE // 64):
            pltpu.sync_copy(zero_v, acc_sp.at[pl.ds(sub * NODES_PER_TILE + k * 64, 64)])
        plsc.subcore_barrier()

        base_blk = (p * NSUB + sub) * NCHUNK

        def super_chunk(ci, carry):
            ib = ib0
            pltpu.sync_copy(edg.at[pl.ds((base_blk + ci) * IBLK, IBLK)], ib)
            cps = {}
            cps[0] = pltpu.async_copy(y_hbm.at[ib.at[0]], rows[0], gs[0])
            for r in range(ACHUNK):
                if r + 1 < ACHUNK:
                    cps[r + 1] = pltpu.async_copy(
                        y_hbm.at[ib.at[r + 1]], rows[(r + 1) % 2], gs[(r + 1) % 2])
                cps[r].wait()
                pltpu.sync_copy(rows[r % 2], acc_sp.at[ib.at[ACHUNK + r]], add=True)
            return carry

        lax.fori_loop(0, NCHUNK, super_chunk, 0)

        plsc.subcore_barrier()
        pltpu.sync_copy(acc_sp.at[pl.ds(sub * NODES_PER_TILE, NODES_PER_TILE)],
                        out.at[pl.ds(p * NPAD + sub * NODES_PER_TILE, NODES_PER_TILE)])


@functools.cache
def _agg_kernel():
    return pl.kernel(
        _agg_body,
        out_type=jax.ShapeDtypeStruct((4 * NPAD, 128), F32),
        mesh=_mesh(),
        scratch_types=[
            pltpu.VMEM((IBLK, 128), jnp.int32),
            pltpu.VMEM((IBLK, 128), jnp.int32),
            pltpu.VMEM((128, 128), F32),
            pltpu.VMEM((128, 128), F32),
            pltpu.VMEM((128, 128), F32),
            pltpu.VMEM((128, 128), F32),
            pltpu.VMEM((64, 128), F32),
            pltpu.VMEM_SHARED((NPAD, 128), F32),
            pltpu.SemaphoreType.DMA,
            pltpu.SemaphoreType.DMA,
            pltpu.SemaphoreType.DMA,
            pltpu.SemaphoreType.DMA,
            pltpu.SemaphoreType.DMA,
            pltpu.SemaphoreType.DMA,
        ],
    )


def _norms_body(din_ref, dout_ref, ns_ref, nd_ref, v_ref):
    din = din_ref[...]
    dout = dout_ref[...]
    ns = lax.rsqrt(jnp.maximum(dout, 1.0))
    nd = lax.rsqrt(jnp.maximum(din, 1.0))
    ns_ref[...] = ns
    nd_ref[...] = nd
    v_ref[...] = ns * din


def _tc_norms(deg_in, deg_out):
    return pl.pallas_call(
        _norms_body,
        out_shape=[jax.ShapeDtypeStruct((NPAD, 1), F32)] * 3,
    )(deg_in, deg_out)


def _h1y_body(t2_ref, nd_ref, ns_ref, w1_ref, b1_ref, w2_ref, y_ref):
    t = t2_ref[0] + t2_ref[1]
    u = t * nd_ref[...]
    h1 = jnp.maximum(u * w1_ref[...] + b1_ref[...], 0.0)
    h1s = h1 * ns_ref[...]
    y_ref[...] = jnp.dot(h1s, w2_ref[...], preferred_element_type=F32)


def _tc_h1y(t2, nd, ns, w1, b1, w2):
    nr = NPAD // 1024
    return pl.pallas_call(
        _h1y_body,
        grid=(nr, 4),
        in_specs=[
            pl.BlockSpec((2, 1024, 1), lambda r, p: (0, r, 0)),
            pl.BlockSpec((1024, 1), lambda r, p: (r, 0)),
            pl.BlockSpec((1024, 1), lambda r, p: (r, 0)),
            pl.BlockSpec((1, HID), lambda r, p: (0, 0)),
            pl.BlockSpec((1, HID), lambda r, p: (0, 0)),
            pl.BlockSpec((HID, 128), lambda r, p: (0, p)),
        ],
        out_specs=pl.BlockSpec((1024, 128), lambda r, p: (p * nr + r, 0)),
        out_shape=jax.ShapeDtypeStruct((4 * NPAD, 128), F32),
    )(t2, nd, ns, w1, b1, w2)


def _pool_body(agg_ref, nd_ref, b2_ref, gid_ref, pooled_ref, cnt_ref):
    r = pl.program_id(0)
    agg = jnp.concatenate([agg_ref[i] for i in range(4)], axis=1)
    h2 = jnp.maximum(agg * nd_ref[...] + b2_ref[...], 0.0)
    gid = gid_ref[...]
    pt = (lax.broadcasted_iota(jnp.int32, (G, 1024), 0)
          == gid).astype(F32)
    pp = jnp.dot(pt, h2, preferred_element_type=F32)
    pc = jnp.dot(pt, jnp.ones((1024, 1), F32), preferred_element_type=F32)

    @pl.when(r == 0)
    def _():
        pooled_ref[...] = pp
        cnt_ref[...] = pc

    @pl.when(r > 0)
    def _():
        pooled_ref[...] += pp
        cnt_ref[...] += pc


def _tc_pool(agg4, nd, b2, gid_row):
    nr = NPAD // 1024
    return pl.pallas_call(
        _pool_body,
        grid=(nr,),
        in_specs=[
            pl.BlockSpec((4, 1024, 128), lambda r: (0, r, 0)),
            pl.BlockSpec((1024, 1), lambda r: (r, 0)),
            pl.BlockSpec((1, HID), lambda r: (0, 0)),
            pl.BlockSpec((1, 1024), lambda r: (0, r)),
        ],
        out_specs=[
            pl.BlockSpec((G, HID), lambda r: (0, 0)),
            pl.BlockSpec((G, 1), lambda r: (0, 0)),
        ],
        out_shape=[
            jax.ShapeDtypeStruct((G, HID), F32),
            jax.ShapeDtypeStruct((G, 1), F32),
        ],
    )(agg4, nd, b2, gid_row)


def _mlp_body(pooled_ref, cnt_ref, w1, b1, w2, b2, w3, b3, w4, b4, w5, b5, out_ref):
    hg = pooled_ref[...] / jnp.maximum(cnt_ref[...], 1.0)
    x = jnp.maximum(jnp.dot(hg, w1[...], preferred_element_type=F32) + b1[...], 0.0)
    x = jnp.maximum(jnp.dot(x, w2[...], preferred_element_type=F32) + b2[...], 0.0)
    x = jnp.maximum(jnp.dot(x, w3[...], preferred_element_type=F32) + b3[...], 0.0)
    x = jnp.maximum(jnp.dot(x, w4[...], preferred_element_type=F32) + b4[...], 0.0)
    logits = jnp.dot(x, w5[...], preferred_element_type=F32) + b5[...]
    m = jnp.max(logits, axis=-1, keepdims=True)
    e = jnp.exp(logits - m)
    out_ref[...] = e / jnp.sum(e, axis=-1, keepdims=True)


def _tc_mlp(pooled, cnt, wc1, bc1, wc2, bc2, wc3, bc3, wc4, bc4, wc5, bc5):
    return pl.pallas_call(
        _mlp_body,
        out_shape=jax.ShapeDtypeStruct((G, 10), F32),
    )(pooled, cnt, wc1, bc1, wc2, bc2, wc3, bc3, wc4, bc4, wc5, bc5)


def kernel(edge_index, graph_ids, W1, b1, W2, b2,
           Wc1, bc1, Wc2, bc2, Wc3, bc3, Wc4, bc4, Wc5, bc5):
    E = edge_index.shape[1]
    src = edge_index[0]
    dst = edge_index[1]
    pad = N + (jnp.arange(EPAD - E, dtype=jnp.int32) % (NPAD - N))
    srcp = jnp.concatenate([src, pad]).reshape(ROWS, 128)
    dstp = jnp.concatenate([dst, pad]).reshape(ROWS, 128)
    srcoff = (srcp[None] + (jnp.arange(4, dtype=jnp.int32) * NPAD)[:, None, None]
              ).reshape(4, NSUB, NCHUNK, ACHUNK, 128)
    dstb = jnp.broadcast_to(dstp.reshape(1, NSUB, NCHUNK, ACHUNK, 128),
                            (4, NSUB, NCHUNK, ACHUNK, 128))
    edg = jnp.concatenate([srcoff, dstb], axis=3).reshape(4 * ROWS * 2, 128)

    deg = _deg_kernel()(srcp, dstp)
    deg_in = deg[:NPAD].reshape(NPAD, 1)
    deg_out = deg[NPAD:].reshape(NPAD, 1)

    ns, nd, v = _tc_norms(deg_in, deg_out)

    t2 = _tsum_kernel()(v.reshape(NPAD), srcp, dstp).reshape(2, NPAD, 1)

    y = _tc_h1y(t2, nd, ns, W1, b1.reshape(1, HID), W2)

    agg = _agg_kernel()(y, edg).reshape(4, NPAD, 128)

    gid_row = jnp.concatenate(
        [graph_ids, jnp.full((NPAD - N,), 1000, jnp.int32)]).reshape(1, NPAD)
    pooled, cnt = _tc_pool(agg, nd, b2.reshape(1, HID), gid_row)

    return _tc_mlp(pooled, cnt,
                   Wc1, bc1.reshape(1, -1), Wc2, bc2.reshape(1, -1),
                   Wc3, bc3.reshape(1, -1), Wc4, bc4.reshape(1, -1),
                   Wc5, bc5.reshape(1, -1))
